# baseline (speedup 1.0000x reference)
import functools

import jax
import jax.numpy as jnp
from jax import lax
from jax.experimental import pallas as pl
from jax.experimental.pallas import tpu as pltpu

from jax.experimental.pallas import tpu_sc as plsc

B = 2
N = 4096
BN = B * N
K = 16
D = 64
HID = 32
S = 16
NB = 5
RES = 64
P2 = RES * RES
PADDING = 0.1

TQ = 256
TP = 256
NW = 32


def _knn_body(q_ref, pt_ref, w4_ref, sb_ref, idx_ref, c0_ref, tri_ref):
    b = pl.program_id(0)
    q = q_ref[0]
    pt = pt_ref[0]
    c0_ref[0] = q @ w4_ref[...] + sb_ref[...]
    psq = jnp.sum(pt * pt, axis=0, keepdims=True)
    qsq = jnp.sum(q * q, axis=1, keepdims=True)
    d2 = qsq - 2.0 * jnp.dot(q, pt, preferred_element_type=jnp.float32) + psq
    colid = lax.broadcasted_iota(jnp.int32, (TQ, N), 1)
    u = lax.bitcast_convert_type(d2, jnp.int32)
    k0 = jnp.where(u < 0, u ^ jnp.int32(0x7FFFFFFF), u)
    key = (k0 & jnp.int32(-4096)) | colid
    big = jnp.int32(0x7FFFFFFF)
    cols = []
    for _ in range(K):
        m = jnp.min(key, axis=1, keepdims=True)
        cols.append(m & jnp.int32(4095))
        key = jnp.where(key == m, big, key)
    idx = jnp.concatenate(cols, axis=1)
    idx_ref[0] = idx + b * N
    norm = (q / (1.0 + PADDING) + 1.0) / 2.0
    ij = jnp.clip(jnp.floor(norm * RES).astype(jnp.int32), 0, RES - 1)
    ix, iy, iz = ij[:, 0], ij[:, 1], ij[:, 2]
    tri = jnp.stack([ix * RES + iy,
                     ix * RES + iz + P2,
                     iy * RES + iz + 2 * P2], axis=0)
    tri_ref[0] = tri


def _knn_call(xyzp, xyzT, w4, sb):
    return pl.pallas_call(
        _knn_body,
        grid=(B, N // TQ),
        in_specs=[
            pl.BlockSpec((1, TQ, 4), lambda b, i: (b, i, 0)),
            pl.BlockSpec((1, 4, N), lambda b, i: (b, 0, 0)),
            pl.BlockSpec((4, D), lambda b, i: (0, 0)),
            pl.BlockSpec((1, D), lambda b, i: (0, 0)),
        ],
        out_specs=[
            pl.BlockSpec((1, TQ, K), lambda b, i: (b, i, 0)),
            pl.BlockSpec((1, TQ, D), lambda b, i: (b, i, 0)),
            pl.BlockSpec((1, 3, TQ), lambda b, i: (b, 0, i)),
        ],
        out_shape=[
            jax.ShapeDtypeStruct((B, N, K), jnp.int32),
            jax.ShapeDtypeStruct((B, N, D), jnp.float32),
            jax.ShapeDtypeStruct((B, 3, N), jnp.int32),
        ],
    )(xyzp, xyzT, w4, sb)


def _sc_gather(table, idxf, row_w):
    M = idxf.shape[0]
    per_w = M // NW
    ch = min(per_w, (1 << 17) // (row_w * 4) // 2 * 2)
    while per_w % ch:
        ch //= 2
    nch = per_w // ch
    mesh = plsc.VectorSubcoreMesh(core_axis_name="c", subcore_axis_name="s")

    @functools.partial(
        pl.kernel, mesh=mesh,
        out_type=jax.ShapeDtypeStruct((M, row_w), jnp.float32),
        compiler_params=pltpu.CompilerParams(use_tc_tiling_on_sc=False),
        scratch_types=[
            pltpu.VMEM((ch,), jnp.int32),
            pltpu.VMEM((2, ch, row_w), jnp.float32),
            pltpu.SemaphoreType.DMA,
            pltpu.SemaphoreType.DMA,
            pltpu.SemaphoreType.DMA,
        ],
    )
    def gk(tab_hbm, idx_hbm, out_hbm, idx_v, rows_v, gsem, osem0, osem1):
        wid = lax.axis_index("s") * 2 + lax.axis_index("c")
        base = wid * per_w
        osems = (osem0, osem1)
        for c in range(nch):
            b = c % 2
            off = base + c * ch
            pltpu.sync_copy(idx_hbm.at[pl.ds(off, ch)], idx_v)
            if c >= 2:
                pltpu.make_async_copy(
                    rows_v.at[b], out_hbm.at[pl.ds(off - 2 * ch, ch)],
                    osems[b]).wait()
            pltpu.async_copy(tab_hbm.at[idx_v], rows_v.at[b], gsem).wait()
            pltpu.async_copy(rows_v.at[b], out_hbm.at[pl.ds(off, ch)], osems[b])
        for c in range(max(nch - 2, 0), nch):
            b = c % 2
            off = base + c * ch
            pltpu.make_async_copy(
                rows_v.at[b], out_hbm.at[pl.ds(off, ch)], osems[b]).wait()

    return gk(table, idxf)


def _prep_body(kx_ref, q_ref, rel_ref):
    kx = kx_ref[...]
    q = q_ref[...]
    rel = kx - q[:, None, :]
    d2 = jnp.sum(rel * rel, axis=2, keepdims=True)
    maxd = jnp.sqrt(jnp.max(d2, axis=1, keepdims=True)) + 1e-9
    rel_ref[...] = rel / maxd


def _prep_call(knn_xyz, xyzp_flat):
    return pl.pallas_call(
        _prep_body,
        grid=(BN // TQ,),
        in_specs=[
            pl.BlockSpec((TQ, K, 4), lambda i: (i, 0, 0)),
            pl.BlockSpec((TQ, 4), lambda i: (i, 0)),
        ],
        out_specs=pl.BlockSpec((TQ, K, 4), lambda i: (i, 0, 0)),
        out_shape=jax.ShapeDtypeStruct((BN, K, 4), jnp.float32),
    )(knn_xyz, xyzp_flat)


NG = TP * K // 128


def _fka_body(c_ref, xnb_ref, rel_ref, w1_ref, b1_ref, w2s_ref, b2s_ref,
              cw_ref, cb_ref, out_ref, macc_ref):
    relf = rel_ref[...].reshape(TP * K, 4)
    h = jnp.maximum(
        jnp.dot(relf, w1_ref[...], preferred_element_type=jnp.float32)
        + b1_ref[...], 0.0)
    ax = jnp.dot(h, w2s_ref[...], preferred_element_type=jnp.float32) \
        + b2s_ref[...]
    rid = lax.broadcasted_iota(jnp.int32, (128, 128), 0)
    cid = lax.broadcasted_iota(jnp.int32, (128, 128), 1)
    mask = (cid % 8) == (rid // 16)
    xnb = xnb_ref[...]
    for g in range(NG):
        bd = jnp.where(mask, ax[g * 128:(g + 1) * 128, :], 0.0)
        xg = xnb[g * 128:(g + 1) * 128, :]
        mg = lax.dot_general(bd, xg, (((0,), (0,)), ((), ())),
                             preferred_element_type=jnp.float32)
        macc_ref[:, g * 8:(g + 1) * 8, :] = mg.reshape(S, 8, D)
    acc = cb_ref[...]
    out = jnp.zeros((TP, D), jnp.float32) + acc
    for s in range(S):
        out = out + jnp.dot(macc_ref[s], cw_ref[s],
                            preferred_element_type=jnp.float32)
    out_ref[...] = jnp.maximum(out, 0.0) + c_ref[...]


def _fka_call(c, xnb, rel, w1p, b1, w2s, b2s, cw, cb):
    return pl.pallas_call(
        _fka_body,
        grid=(BN // TP,),
        in_specs=[
            pl.BlockSpec((TP, D), lambda i: (i, 0)),
            pl.BlockSpec((TP * K, D), lambda i: (i, 0)),
            pl.BlockSpec((TP, K, 4), lambda i: (i, 0, 0)),
            pl.BlockSpec((4, HID), lambda i: (0, 0)),
            pl.BlockSpec((1, HID), lambda i: (0, 0)),
            pl.BlockSpec((HID, 128), lambda i: (0, 0)),
            pl.BlockSpec((1, 128), lambda i: (0, 0)),
            pl.BlockSpec((S, D, D), lambda i: (0, 0, 0)),
            pl.BlockSpec((1, D), lambda i: (0, 0)),
        ],
        out_specs=pl.BlockSpec((TP, D), lambda i: (i, 0)),
        out_shape=jax.ShapeDtypeStruct((BN, D), jnp.float32),
        scratch_shapes=[pltpu.VMEM((S, TP, D), jnp.float32)],
    )(c, xnb, rel, w1p, b1, w2s, b2s, cw, cb)


def _sc_scatter(cfeat, tidx, zeros64, zeros16):
    npts = N // 16
    rows_w = 3 * P2 // 16
    mesh = plsc.VectorSubcoreMesh(core_axis_name="c", subcore_axis_name="s")

    @functools.partial(
        pl.kernel, mesh=mesh,
        out_type=[
            jax.ShapeDtypeStruct((B, 3 * P2, D), jnp.float32),
            jax.ShapeDtypeStruct((B, 3 * P2, 16), jnp.float32),
        ],
        compiler_params=pltpu.CompilerParams(use_tc_tiling_on_sc=False),
        scratch_types=[
            pltpu.VMEM((npts, D), jnp.float32),
            pltpu.VMEM((npts,), jnp.int32),
            pltpu.VMEM((npts, 16), jnp.float32),
            pltpu.VMEM_SHARED((3 * P2, D), jnp.float32),
            pltpu.VMEM_SHARED((3 * P2, 16), jnp.float32),
        ],
    )
    def sk(cf_hbm, ti_hbm, z64_hbm, z16_hbm, acc_hbm, cnt_hbm,
           feat_v, idx_v, ones_v, acc_sh, cnt_sh):
        b = lax.axis_index("c")
        s = lax.axis_index("s")
        pltpu.sync_copy(z64_hbm.at[pl.ds(s * rows_w, rows_w)],
                        acc_sh.at[pl.ds(s * rows_w, rows_w)])
        pltpu.sync_copy(z16_hbm.at[pl.ds(s * rows_w, rows_w)],
                        cnt_sh.at[pl.ds(s * rows_w, rows_w)])
        pltpu.sync_copy(cf_hbm.at[pl.ds(b * N + s * npts, npts)], feat_v)

        def fill(i, _):
            ones_v[i] = jnp.full((16,), 1.0, jnp.float32)
            return 0
        lax.fori_loop(0, npts, fill, 0)
        plsc.subcore_barrier()
        for p in range(3):
            pltpu.sync_copy(ti_hbm.at[b, p, pl.ds(s * npts, npts)], idx_v)
            pltpu.sync_copy(feat_v, acc_sh.at[idx_v], add=True)
            pltpu.sync_copy(ones_v, cnt_sh.at[idx_v], add=True)
        plsc.subcore_barrier()
        pltpu.sync_copy(acc_sh.at[pl.ds(s * rows_w, rows_w)],
                        acc_hbm.at[b, pl.ds(s * rows_w, rows_w)])
        pltpu.sync_copy(cnt_sh.at[pl.ds(s * rows_w, rows_w)],
                        cnt_hbm.at[b, pl.ds(s * rows_w, rows_w)])

    return sk(cfeat, tidx, zeros64, zeros16)


def _fin_body(a_ref, n_ref, o_ref):
    cnt = n_ref[...][:, 0:1]
    o_ref[...] = a_ref[...] / jnp.maximum(cnt, 1.0)


def _fin_call(acc, cnt):
    rows = B * 3 * P2
    tile = 2048
    return pl.pallas_call(
        _fin_body,
        grid=(rows // tile,),
        in_specs=[
            pl.BlockSpec((tile, D), lambda i: (i, 0)),
            pl.BlockSpec((tile, 16), lambda i: (i, 0)),
        ],
        out_specs=pl.BlockSpec((tile, D), lambda i: (i, 0)),
        out_shape=jax.ShapeDtypeStruct((rows, D), jnp.float32),
    )(acc, cnt)


def kernel(xyz, stem_w, stem_b, fka1_w, fka1_b, fka2_w, fka2_b, conv_w, conv_b):
    f32 = jnp.float32
    xyzp = jnp.pad(xyz, ((0, 0), (0, 0), (0, 1)))
    xyzT = jnp.swapaxes(xyzp, 1, 2)
    w4 = jnp.pad(stem_w, ((0, 1), (0, 0)))
    sb = stem_b[None, :]

    knn_flat, c0, tidx = _knn_call(xyzp, xyzT, w4, sb)
    idxf = knn_flat.reshape(BN * K)

    xyz16 = jnp.pad(xyzp.reshape(BN, 4), ((0, 0), (0, 12)))
    knn_xyz = _sc_gather(xyz16, idxf, 16).reshape(BN, K, 16)[:, :, :4]
    rel = _prep_call(knn_xyz, xyzp.reshape(BN, 4))

    w1p = jnp.pad(fka1_w, ((0, 0), (0, 1), (0, 0)))
    rep = jnp.repeat(jnp.eye(S, dtype=f32), 8, axis=1)
    w2s = jnp.einsum('bhs,sc->bhc', fka2_w, rep) / K
    b2s = jnp.einsum('bs,sc->bc', fka2_b, rep) / K

    c = c0.reshape(BN, D)
    for i in range(NB):
        xnb = _sc_gather(c, idxf, D)
        c = _fka_call(c, xnb, rel, w1p[i], fka1_b[i][None, :],
                      w2s[i], b2s[i][None, :], conv_w[i], conv_b[i][None, :])

    zeros64 = jnp.zeros((3 * P2, D), f32)
    zeros16 = jnp.zeros((3 * P2, 16), f32)
    acc, cnt = _sc_scatter(c, tidx, zeros64, zeros16)
    tri = _fin_call(acc.reshape(B * 3 * P2, D), cnt.reshape(B * 3 * P2, 16))
    tri_feat = tri.reshape(B, 3, RES, RES, D)
    return (xyz, c.reshape(B, N, D), tri_feat)


# speedup vs baseline: 1.1287x; 1.1287x over previous
import functools

import jax
import jax.numpy as jnp
from jax import lax
from jax.experimental import pallas as pl
from jax.experimental.pallas import tpu as pltpu

from jax.experimental.pallas import tpu_sc as plsc

B = 2
N = 4096
BN = B * N
K = 16
D = 64
HID = 32
S = 16
NB = 5
RES = 64
P2 = RES * RES
PADDING = 0.1

TQ = 256
TP = 256
NW = 32


def _knn_body(q_ref, pt_ref, w4_ref, sb_ref, idx_ref, c0_ref, tri_ref):
    b = pl.program_id(0)
    q = q_ref[0]
    pt = pt_ref[0]
    c0_ref[0] = q @ w4_ref[...] + sb_ref[...]
    psq = jnp.sum(pt * pt, axis=0, keepdims=True)
    qsq = jnp.sum(q * q, axis=1, keepdims=True)
    d2 = qsq - 2.0 * jnp.dot(q, pt, preferred_element_type=jnp.float32) + psq
    colid = lax.broadcasted_iota(jnp.int32, (TQ, N), 1)
    u = lax.bitcast_convert_type(d2, jnp.int32)
    k0 = jnp.where(u < 0, u ^ jnp.int32(0x7FFFFFFF), u)
    key = (k0 & jnp.int32(-4096)) | colid
    big = jnp.int32(0x7FFFFFFF)
    cols = []
    for _ in range(K):
        m = jnp.min(key, axis=1, keepdims=True)
        cols.append(m & jnp.int32(4095))
        key = jnp.where(key == m, big, key)
    idx = jnp.concatenate(cols, axis=1)
    idx_ref[0] = idx + b * N
    norm = (q / (1.0 + PADDING) + 1.0) / 2.0
    ij = jnp.clip(jnp.floor(norm * RES).astype(jnp.int32), 0, RES - 1)
    ix, iy, iz = ij[:, 0], ij[:, 1], ij[:, 2]
    tri = jnp.stack([ix * RES + iy,
                     ix * RES + iz + P2,
                     iy * RES + iz + 2 * P2], axis=0)
    tri_ref[0] = tri


def _knn_call(xyzp, xyzT, w4, sb):
    return pl.pallas_call(
        _knn_body,
        grid=(B, N // TQ),
        in_specs=[
            pl.BlockSpec((1, TQ, 4), lambda b, i: (b, i, 0)),
            pl.BlockSpec((1, 4, N), lambda b, i: (b, 0, 0)),
            pl.BlockSpec((4, 128), lambda b, i: (0, 0)),
            pl.BlockSpec((1, 128), lambda b, i: (0, 0)),
        ],
        out_specs=[
            pl.BlockSpec((1, TQ, K), lambda b, i: (b, i, 0)),
            pl.BlockSpec((1, TQ, 128), lambda b, i: (b, i, 0)),
            pl.BlockSpec((1, 3, TQ), lambda b, i: (b, 0, i)),
        ],
        out_shape=[
            jax.ShapeDtypeStruct((B, N, K), jnp.int32),
            jax.ShapeDtypeStruct((B, N, 128), jnp.float32),
            jax.ShapeDtypeStruct((B, 3, N), jnp.int32),
        ],
    )(xyzp, xyzT, w4, sb)


def _sc_gather(table, idxf, row_w, tc_tiling):
    M = idxf.shape[0]
    per_w = M // NW
    ch = min(per_w, (1 << 17) // (row_w * 4))
    while per_w % ch:
        ch //= 2
    nch = per_w // ch
    mesh = plsc.VectorSubcoreMesh(core_axis_name="c", subcore_axis_name="s")

    @functools.partial(
        pl.kernel, mesh=mesh,
        out_type=jax.ShapeDtypeStruct((M, row_w), jnp.float32),
        compiler_params=pltpu.CompilerParams(use_tc_tiling_on_sc=tc_tiling),
        scratch_types=[
            pltpu.VMEM((ch,), jnp.int32),
            pltpu.VMEM((2, ch, row_w), jnp.float32),
            pltpu.SemaphoreType.DMA,
            pltpu.SemaphoreType.DMA,
            pltpu.SemaphoreType.DMA,
        ],
    )
    def gk(tab_hbm, idx_hbm, out_hbm, idx_v, rows_v, gsem, osem0, osem1):
        wid = lax.axis_index("s") * 2 + lax.axis_index("c")
        base = wid * per_w
        osems = (osem0, osem1)
        for c in range(nch):
            b = c % 2
            off = base + c * ch
            pltpu.sync_copy(idx_hbm.at[pl.ds(off, ch)], idx_v)
            if c >= 2:
                pltpu.make_async_copy(
                    rows_v.at[b], out_hbm.at[pl.ds(off - 2 * ch, ch)],
                    osems[b]).wait()
            pltpu.async_copy(tab_hbm.at[idx_v], rows_v.at[b], gsem).wait()
            pltpu.async_copy(rows_v.at[b], out_hbm.at[pl.ds(off, ch)], osems[b])
        for c in range(max(nch - 2, 0), nch):
            b = c % 2
            off = base + c * ch
            pltpu.make_async_copy(
                rows_v.at[b], out_hbm.at[pl.ds(off, ch)], osems[b]).wait()

    return gk(table, idxf)


def _prep_body(kx_ref, q_ref, rel_ref):
    kx = kx_ref[...].reshape(TQ, K, 4)
    q = q_ref[...]
    rel = kx - q[:, None, :]
    d2 = jnp.sum(rel * rel, axis=2, keepdims=True)
    maxd = jnp.sqrt(jnp.max(d2, axis=1, keepdims=True)) + 1e-9
    rel_ref[...] = rel / maxd


def _prep_call(knn_xyz, xyzp_flat):
    return pl.pallas_call(
        _prep_body,
        grid=(BN // TQ,),
        in_specs=[
            pl.BlockSpec((TQ * K, 4), lambda i: (i, 0)),
            pl.BlockSpec((TQ, 4), lambda i: (i, 0)),
        ],
        out_specs=pl.BlockSpec((TQ, K, 4), lambda i: (i, 0, 0)),
        out_shape=jax.ShapeDtypeStruct((BN, K, 4), jnp.float32),
    )(knn_xyz, xyzp_flat)


NG = TP * K // 128


def _fka_body(c_ref, xnb_ref, rel_ref, w1_ref, b1_ref, w2s_ref, b2s_ref,
              cw_ref, cb_ref, out_ref, macc_ref):
    relf = rel_ref[...].reshape(TP * K, 4)
    h = jnp.maximum(
        jnp.dot(relf, w1_ref[...], preferred_element_type=jnp.float32)
        + b1_ref[...], 0.0)
    ax = jnp.dot(h, w2s_ref[...], preferred_element_type=jnp.float32) \
        + b2s_ref[...]
    rid = lax.broadcasted_iota(jnp.int32, (128, 128), 0)
    cid = lax.broadcasted_iota(jnp.int32, (128, 128), 1)
    mask = (cid % 8) == (rid // 16)
    xnb = xnb_ref[...][:, :D]
    for g in range(NG):
        bd = jnp.where(mask, ax[g * 128:(g + 1) * 128, :], 0.0)
        xg = xnb[g * 128:(g + 1) * 128, :]
        mg = lax.dot_general(bd, xg, (((0,), (0,)), ((), ())),
                             preferred_element_type=jnp.float32)
        macc_ref[:, g * 8:(g + 1) * 8, :] = mg.reshape(S, 8, D)
    acc = cb_ref[...]
    out = jnp.zeros((TP, D), jnp.float32) + acc
    for s in range(S):
        out = out + jnp.dot(macc_ref[s], cw_ref[s],
                            preferred_element_type=jnp.float32)
    res = jnp.maximum(out, 0.0) + c_ref[...][:, :D]
    out_ref[...] = jnp.concatenate([res, jnp.zeros((TP, 128 - D), jnp.float32)],
                                   axis=1)


def _fka_call(c, xnb, rel, w1p, b1, w2s, b2s, cw, cb):
    return pl.pallas_call(
        _fka_body,
        grid=(BN // TP,),
        in_specs=[
            pl.BlockSpec((TP, 128), lambda i: (i, 0)),
            pl.BlockSpec((TP * K, 128), lambda i: (i, 0)),
            pl.BlockSpec((TP, K, 4), lambda i: (i, 0, 0)),
            pl.BlockSpec((4, HID), lambda i: (0, 0)),
            pl.BlockSpec((1, HID), lambda i: (0, 0)),
            pl.BlockSpec((HID, 128), lambda i: (0, 0)),
            pl.BlockSpec((1, 128), lambda i: (0, 0)),
            pl.BlockSpec((S, D, D), lambda i: (0, 0, 0)),
            pl.BlockSpec((1, D), lambda i: (0, 0)),
        ],
        out_specs=pl.BlockSpec((TP, 128), lambda i: (i, 0)),
        out_shape=jax.ShapeDtypeStruct((BN, 128), jnp.float32),
        scratch_shapes=[pltpu.VMEM((S, TP, D), jnp.float32)],
    )(c, xnb, rel, w1p, b1, w2s, b2s, cw, cb)


def _sc_scatter(cfeat, tidx, zeros64, zeros16):
    npts = N // 16
    rows_w = 3 * P2 // 16
    mesh = plsc.VectorSubcoreMesh(core_axis_name="c", subcore_axis_name="s")

    @functools.partial(
        pl.kernel, mesh=mesh,
        out_type=[
            jax.ShapeDtypeStruct((B, 3 * P2, D), jnp.float32),
            jax.ShapeDtypeStruct((B, 3 * P2, 16), jnp.float32),
        ],
        compiler_params=pltpu.CompilerParams(use_tc_tiling_on_sc=False),
        scratch_types=[
            pltpu.VMEM((npts, D), jnp.float32),
            pltpu.VMEM((npts,), jnp.int32),
            pltpu.VMEM((npts, 16), jnp.float32),
            pltpu.VMEM_SHARED((3 * P2, D), jnp.float32),
            pltpu.VMEM_SHARED((3 * P2, 16), jnp.float32),
        ],
    )
    def sk(cf_hbm, ti_hbm, z64_hbm, z16_hbm, acc_hbm, cnt_hbm,
           feat_v, idx_v, ones_v, acc_sh, cnt_sh):
        b = lax.axis_index("c")
        s = lax.axis_index("s")
        pltpu.sync_copy(z64_hbm.at[pl.ds(s * rows_w, rows_w)],
                        acc_sh.at[pl.ds(s * rows_w, rows_w)])
        pltpu.sync_copy(z16_hbm.at[pl.ds(s * rows_w, rows_w)],
                        cnt_sh.at[pl.ds(s * rows_w, rows_w)])
        pltpu.sync_copy(cf_hbm.at[pl.ds(b * N + s * npts, npts)], feat_v)

        def fill(i, _):
            ones_v[i] = jnp.full((16,), 1.0, jnp.float32)
            return 0
        lax.fori_loop(0, npts, fill, 0)
        plsc.subcore_barrier()
        for p in range(3):
            pltpu.sync_copy(ti_hbm.at[b, p, pl.ds(s * npts, npts)], idx_v)
            pltpu.sync_copy(feat_v, acc_sh.at[idx_v], add=True)
            pltpu.sync_copy(ones_v, cnt_sh.at[idx_v], add=True)
        plsc.subcore_barrier()
        pltpu.sync_copy(acc_sh.at[pl.ds(s * rows_w, rows_w)],
                        acc_hbm.at[b, pl.ds(s * rows_w, rows_w)])
        pltpu.sync_copy(cnt_sh.at[pl.ds(s * rows_w, rows_w)],
                        cnt_hbm.at[b, pl.ds(s * rows_w, rows_w)])

    return sk(cfeat, tidx, zeros64, zeros16)


def _fin_body(a_ref, n_ref, o_ref):
    cnt = n_ref[...][:, 0:1]
    o_ref[...] = a_ref[...] / jnp.maximum(cnt, 1.0)


def _fin_call(acc, cnt):
    rows = B * 3 * P2
    tile = 2048
    return pl.pallas_call(
        _fin_body,
        grid=(rows // tile,),
        in_specs=[
            pl.BlockSpec((tile, D), lambda i: (i, 0)),
            pl.BlockSpec((tile, 16), lambda i: (i, 0)),
        ],
        out_specs=pl.BlockSpec((tile, D), lambda i: (i, 0)),
        out_shape=jax.ShapeDtypeStruct((rows, D), jnp.float32),
    )(acc, cnt)


def kernel(xyz, stem_w, stem_b, fka1_w, fka1_b, fka2_w, fka2_b, conv_w, conv_b):
    f32 = jnp.float32
    xyzp = jnp.pad(xyz, ((0, 0), (0, 0), (0, 1)))
    xyzT = jnp.swapaxes(xyzp, 1, 2)
    w4 = jnp.pad(stem_w, ((0, 1), (0, 64)))
    sb = jnp.pad(stem_b, (0, 64))[None, :]

    knn_flat, c0, tidx = _knn_call(xyzp, xyzT, w4, sb)
    idxf = knn_flat.reshape(BN * K)

    knn_xyz = _sc_gather(xyzp.reshape(BN, 4), idxf, 4, False)
    rel = _prep_call(knn_xyz, xyzp.reshape(BN, 4))

    w1p = jnp.pad(fka1_w, ((0, 0), (0, 1), (0, 0)))
    rep = jnp.repeat(jnp.eye(S, dtype=f32), 8, axis=1)
    w2s = jnp.einsum('bhs,sc->bhc', fka2_w, rep) / K
    b2s = jnp.einsum('bs,sc->bc', fka2_b, rep) / K

    c = c0.reshape(BN, 128)
    for i in range(NB):
        xnb = _sc_gather(c, idxf, 128, True)
        c = _fka_call(c, xnb, rel, w1p[i], fka1_b[i][None, :],
                      w2s[i], b2s[i][None, :], conv_w[i], conv_b[i][None, :])

    cd = c[:, :D]
    zeros64 = jnp.zeros((3 * P2, D), f32)
    zeros16 = jnp.zeros((3 * P2, 16), f32)
    acc, cnt = _sc_scatter(cd, tidx, zeros64, zeros16)
    tri = _fin_call(acc.reshape(B * 3 * P2, D), cnt.reshape(B * 3 * P2, 16))
    tri_feat = tri.reshape(B, 3, RES, RES, D)
    return (xyz, cd.reshape(B, N, D), tri_feat)
